# baseline (speedup 1.0000x reference)
import jax
import jax.numpy as jnp
from jax import lax
from jax.experimental import pallas as pl
from jax.experimental.pallas import tpu as pltpu
from jax.experimental.pallas import tpu_sc as plsc

N = 10000
NP = 10240
E = 320000
EP = 327680
FIN = 128
D1 = 128
ACC_W = 144
NCORES = 2
NSUB = 16
NTILES = NCORES * NSUB
EDGES_PER_TILE = EP // NTILES
CHUNK1 = 80
EPT_C0 = 7040
EPT_C1 = 13440
NC0 = EPT_C0 // CHUNK1
NC1 = EPT_C1 // CHUNK1
CHUNK2 = 128
NCHUNK2 = EDGES_PER_TILE // CHUNK2
ROWS_PER_TILE = NP // NSUB

_HI = jax.lax.Precision.HIGHEST


def _lrelu_exp(v):
    return jnp.exp(jnp.maximum(v, 0.0) + 0.2 * jnp.minimum(v, 0.0))


def _lane_bcast(v, lane):
    idx = jnp.full((16, 1), lane, jnp.int32)
    dn = lax.GatherDimensionNumbers(
        offset_dims=(), collapsed_slice_dims=(0,), start_index_map=(0,))
    return lax.gather(v, idx, dn, (1,),
                      mode=lax.GatherScatterMode.PROMISE_IN_BOUNDS)


def _stage_a_body(x_ref, w1_ref, a1s_ref, a1d_ref, hx_ref, ad_ref):
    h = lax.dot_general(x_ref[...], w1_ref[...], (((1,), (0,)), ((), ())),
                        precision=_HI, preferred_element_type=jnp.float32)
    hx_ref[:, :D1] = h
    hx_ref[:, D1:ACC_W] = lax.dot_general(
        h, a1s_ref[...], (((1,), (0,)), ((), ())),
        precision=_HI, preferred_element_type=jnp.float32)
    ad_ref[...] = lax.dot_general(h, a1d_ref[...], (((1,), (0,)), ((), ())),
                                  precision=_HI,
                                  preferred_element_type=jnp.float32)


def _stage_a(xp, W1, A1s, A1d):
    mb = 2048
    grid = (NP // mb,)
    return pl.pallas_call(
        _stage_a_body,
        grid=grid,
        in_specs=[
            pl.BlockSpec((mb, FIN), lambda i: (i, 0)),
            pl.BlockSpec((FIN, D1), lambda i: (0, 0)),
            pl.BlockSpec((D1, 16), lambda i: (0, 0)),
            pl.BlockSpec((D1, 16), lambda i: (0, 0)),
        ],
        out_specs=[
            pl.BlockSpec((mb, ACC_W), lambda i: (i, 0)),
            pl.BlockSpec((mb, 16), lambda i: (i, 0)),
        ],
        out_shape=[
            jax.ShapeDtypeStruct((NP, ACC_W), jnp.float32),
            jax.ShapeDtypeStruct((NP, 16), jnp.float32),
        ],
    )(xp, W1, A1s, A1d)


def _sc1_body(src_hbm, dst_hbm, hx_hbm, ad_hbm, out_hbm,
              sidxA, didxA, sidxB, didxB, adrA, adrB, msgA, msgB,
              acc, gA, gB, sA, sB):
    c = lax.axis_index("c")
    s = lax.axis_index("s")
    wid = c * NSUB + s
    zero16 = jnp.zeros((16,), jnp.float32)

    @pl.loop(0, CHUNK1)
    def _zrow(i):
        @pl.loop(0, ACC_W // 16)
        def _zcol(j):
            msgA[i, pl.ds(j * 16, 16)] = zero16

    @pl.loop(0, ROWS_PER_TILE // CHUNK1)
    def _zacc(i):
        pltpu.sync_copy(msgA, acc.at[pl.ds(s * ROWS_PER_TILE + i * CHUNK1,
                                           CHUNK1)])

    plsc.subcore_barrier()

    def pipeline(base_w, nchunk):
        def load_idx(i, sidx, didx):
            b = base_w + i * CHUNK1
            pltpu.sync_copy(src_hbm.at[pl.ds(b, CHUNK1)], sidx)
            pltpu.sync_copy(dst_hbm.at[pl.ds(b, CHUNK1)], didx)

        def start_gather(sidx, didx, adr, msg, sem):
            pltpu.async_copy(hx_hbm.at[sidx], msg, sem)
            pltpu.async_copy(ad_hbm.at[didx], adr, sem)

        def wait_gather(sidx, didx, adr, msg, sem):
            pltpu.make_async_copy(hx_hbm.at[sidx], msg, sem).wait()
            pltpu.make_async_copy(ad_hbm.at[didx], adr, sem).wait()

        def compute(adr, msg):
            @plsc.parallel_loop(0, CHUNK1, 1, unroll=2)
            def _edge(e):
                v = msg[e, pl.ds(D1, 16)] + adr[e, :]
                w = _lrelu_exp(v)
                msg[e, pl.ds(D1, 16)] = w
                for hh in range(8):
                    bc = _lane_bcast(w, hh)
                    msg[e, pl.ds(hh * 16, 16)] = msg[e, pl.ds(hh * 16, 16)] * bc

        def start_scatter(msg, didx, sem):
            pltpu.async_copy(msg, acc.at[didx], sem, add=True)

        def wait_scatter(msg, didx, sem):
            pltpu.make_async_copy(msg, acc.at[didx], sem).wait()

        load_idx(0, sidxA, didxA)
        start_gather(sidxA, didxA, adrA, msgA, gA)
        load_idx(1, sidxB, didxB)
        start_gather(sidxB, didxB, adrB, msgB, gB)

        @pl.loop(0, nchunk // 2 - 1)
        def _round(r):
            i = 2 * r
            wait_gather(sidxA, didxA, adrA, msgA, gA)
            compute(adrA, msgA)
            start_scatter(msgA, didxA, sA)
            wait_gather(sidxB, didxB, adrB, msgB, gB)
            compute(adrB, msgB)
            start_scatter(msgB, didxB, sB)
            wait_scatter(msgA, didxA, sA)
            load_idx(i + 2, sidxA, didxA)
            start_gather(sidxA, didxA, adrA, msgA, gA)
            wait_scatter(msgB, didxB, sB)
            load_idx(i + 3, sidxB, didxB)
            start_gather(sidxB, didxB, adrB, msgB, gB)

        wait_gather(sidxA, didxA, adrA, msgA, gA)
        compute(adrA, msgA)
        start_scatter(msgA, didxA, sA)
        wait_gather(sidxB, didxB, adrB, msgB, gB)
        compute(adrB, msgB)
        start_scatter(msgB, didxB, sB)
        wait_scatter(msgA, didxA, sA)
        wait_scatter(msgB, didxB, sB)

    @pl.when(c == 0)
    def _p0():
        pipeline(s * EPT_C0, NC0)

    @pl.when(c == 1)
    def _p1():
        pipeline(EPT_C0 * NSUB + s * EPT_C1, NC1)

    plsc.subcore_barrier()
    row0 = s * ROWS_PER_TILE
    src_slice = acc.at[pl.ds(row0, ROWS_PER_TILE)]

    @pl.when(c == 0)
    def _w0():
        pltpu.sync_copy(src_slice, out_hbm.at[0, pl.ds(row0, ROWS_PER_TILE)])

    @pl.when(c == 1)
    def _w1():
        pltpu.sync_copy(src_slice, out_hbm.at[1, pl.ds(row0, ROWS_PER_TILE)])


def _sc1(src, dst, HX, AD):
    mesh = plsc.VectorSubcoreMesh(core_axis_name="c", subcore_axis_name="s")
    return pl.kernel(
        _sc1_body,
        mesh=mesh,
        compiler_params=pltpu.CompilerParams(use_tc_tiling_on_sc=False, needs_layout_passes=False),
        out_type=jax.ShapeDtypeStruct((2, NP, ACC_W), jnp.float32),
        scratch_types=[
            pltpu.VMEM((CHUNK1,), jnp.int32),
            pltpu.VMEM((CHUNK1,), jnp.int32),
            pltpu.VMEM((CHUNK1,), jnp.int32),
            pltpu.VMEM((CHUNK1,), jnp.int32),
            pltpu.VMEM((CHUNK1, 16), jnp.float32),
            pltpu.VMEM((CHUNK1, 16), jnp.float32),
            pltpu.VMEM((CHUNK1, ACC_W), jnp.float32),
            pltpu.VMEM((CHUNK1, ACC_W), jnp.float32),
            pltpu.VMEM_SHARED((NP, ACC_W), jnp.float32),
            pltpu.SemaphoreType.DMA,
            pltpu.SemaphoreType.DMA,
            pltpu.SemaphoreType.DMA,
            pltpu.SemaphoreType.DMA,
        ],
    )(src, dst, HX, AD)


def _stage_b_body(p0_ref, p1_ref, hx_ref, ad_ref, b1_ref, w2_ref,
                  a2w_ref, rsel_ref, t2_ref, a2f_ref):
    accf = p0_ref[...] + p1_ref[...]
    hx = hx_ref[...]
    ws = _lrelu_exp(hx[:, D1:ACC_W] + ad_ref[...])
    rsel = rsel_ref[...]
    ws_wide = lax.dot_general(ws, rsel, (((1,), (0,)), ((), ())),
                              precision=_HI, preferred_element_type=jnp.float32)
    den_wide = lax.dot_general(accf[:, D1:ACC_W] + ws, rsel,
                               (((1,), (0,)), ((), ())),
                               precision=_HI, preferred_element_type=jnp.float32)
    num = accf[:, :D1] + ws_wide * hx[:, :D1]
    out1 = num / den_wide + b1_ref[...]
    z = jnp.where(out1 > 0, out1, jnp.exp(jnp.minimum(out1, 0.0)) - 1.0)
    h2 = lax.dot_general(z, w2_ref[...], (((1,), (0,)), ((), ())),
                         precision=_HI, preferred_element_type=jnp.float32)
    lane = lax.broadcasted_iota(jnp.int32, (1, 16), 1)
    t2_ref[...] = h2 + jnp.where(lane == 10, 1.0, 0.0)
    a2f_ref[...] = lax.dot_general(h2, a2w_ref[...], (((1,), (0,)), ((), ())),
                                   precision=_HI,
                                   preferred_element_type=jnp.float32)


def _stage_b(P0, P1, HX, AD, b1m, W2p, A2W, Rsel):
    mb = 2048
    grid = (NP // mb,)
    return pl.pallas_call(
        _stage_b_body,
        grid=grid,
        in_specs=[
            pl.BlockSpec((mb, ACC_W), lambda i: (i, 0)),
            pl.BlockSpec((mb, ACC_W), lambda i: (i, 0)),
            pl.BlockSpec((mb, ACC_W), lambda i: (i, 0)),
            pl.BlockSpec((mb, 16), lambda i: (i, 0)),
            pl.BlockSpec((1, D1), lambda i: (0, 0)),
            pl.BlockSpec((D1, 16), lambda i: (0, 0)),
            pl.BlockSpec((16, 16), lambda i: (0, 0)),
            pl.BlockSpec((16, D1), lambda i: (0, 0)),
        ],
        out_specs=[
            pl.BlockSpec((mb, 16), lambda i: (i, 0)),
            pl.BlockSpec((mb, 16), lambda i: (i, 0)),
        ],
        out_shape=[
            jax.ShapeDtypeStruct((NP, 16), jnp.float32),
            jax.ShapeDtypeStruct((NP, 16), jnp.float32),
        ],
    )(P0, P1, HX, AD, b1m, W2p, A2W, Rsel)


def _sc2_body(src_hbm, dst_hbm, t2_hbm, as2_hbm, ad2_hbm, out_hbm,
              sidxA, didxA, sidxB, didxB, msgA, msgB, as2l, ad2l,
              acc, gA, gB, sA, sB):
    c = lax.axis_index("c")
    s = lax.axis_index("s")
    wid = c * NSUB + s
    pltpu.sync_copy(as2_hbm, as2l)
    pltpu.sync_copy(ad2_hbm, ad2l)
    zero16 = jnp.zeros((16,), jnp.float32)

    @pl.loop(0, CHUNK2)
    def _zrow(i):
        msgA[i, :] = zero16

    @pl.loop(0, ROWS_PER_TILE // CHUNK2)
    def _zacc(i):
        pltpu.sync_copy(msgA, acc.at[pl.ds(s * ROWS_PER_TILE + i * CHUNK2,
                                           CHUNK2)])

    plsc.subcore_barrier()

    base_w = wid * EDGES_PER_TILE

    def load_idx(i, sidx, didx):
        b = base_w + i * CHUNK2
        pltpu.sync_copy(src_hbm.at[pl.ds(b, CHUNK2)], sidx)
        pltpu.sync_copy(dst_hbm.at[pl.ds(b, CHUNK2)], didx)

    def start_gather(sidx, msg, sem):
        pltpu.async_copy(t2_hbm.at[sidx], msg, sem)

    def wait_gather(sidx, msg, sem):
        pltpu.make_async_copy(t2_hbm.at[sidx], msg, sem).wait()

    def compute(sidx, didx, msg):
        @plsc.parallel_loop(0, CHUNK2 // 16, 1, unroll=2)
        def _grp(g):
            sv = sidx[pl.ds(g * 16, 16)]
            dv = didx[pl.ds(g * 16, 16)]
            av = plsc.load_gather(as2l, [sv])
            bv = plsc.load_gather(ad2l, [dv])
            w2 = _lrelu_exp(av + bv)
            for j in range(16):
                bc = _lane_bcast(w2, j)
                msg[g * 16 + j, :] = msg[g * 16 + j, :] * bc

    def start_scatter(msg, didx, sem):
        pltpu.async_copy(msg, acc.at[didx], sem, add=True)

    def wait_scatter(msg, didx, sem):
        pltpu.make_async_copy(msg, acc.at[didx], sem).wait()

    load_idx(0, sidxA, didxA)
    start_gather(sidxA, msgA, gA)
    load_idx(1, sidxB, didxB)
    start_gather(sidxB, msgB, gB)

    @pl.loop(0, NCHUNK2 // 2 - 1)
    def _round(r):
        i = 2 * r
        wait_gather(sidxA, msgA, gA)
        compute(sidxA, didxA, msgA)
        start_scatter(msgA, didxA, sA)
        wait_gather(sidxB, msgB, gB)
        compute(sidxB, didxB, msgB)
        start_scatter(msgB, didxB, sB)
        wait_scatter(msgA, didxA, sA)
        load_idx(i + 2, sidxA, didxA)
        start_gather(sidxA, msgA, gA)
        wait_scatter(msgB, didxB, sB)
        load_idx(i + 3, sidxB, didxB)
        start_gather(sidxB, msgB, gB)

    wait_gather(sidxA, msgA, gA)
    compute(sidxA, didxA, msgA)
    start_scatter(msgA, didxA, sA)
    wait_gather(sidxB, msgB, gB)
    compute(sidxB, didxB, msgB)
    start_scatter(msgB, didxB, sB)
    wait_scatter(msgA, didxA, sA)
    wait_scatter(msgB, didxB, sB)

    plsc.subcore_barrier()
    row0 = s * ROWS_PER_TILE
    src_slice = acc.at[pl.ds(row0, ROWS_PER_TILE)]

    @pl.when(c == 0)
    def _w0():
        pltpu.sync_copy(src_slice, out_hbm.at[0, pl.ds(row0, ROWS_PER_TILE)])

    @pl.when(c == 1)
    def _w1():
        pltpu.sync_copy(src_slice, out_hbm.at[1, pl.ds(row0, ROWS_PER_TILE)])


def _sc2(src, dst, T2, as2f, ad2f):
    mesh = plsc.VectorSubcoreMesh(core_axis_name="c", subcore_axis_name="s")
    return pl.kernel(
        _sc2_body,
        mesh=mesh,
        compiler_params=pltpu.CompilerParams(use_tc_tiling_on_sc=False, needs_layout_passes=False),
        out_type=jax.ShapeDtypeStruct((2, NP, 16), jnp.float32),
        scratch_types=[
            pltpu.VMEM((CHUNK2,), jnp.int32),
            pltpu.VMEM((CHUNK2,), jnp.int32),
            pltpu.VMEM((CHUNK2,), jnp.int32),
            pltpu.VMEM((CHUNK2,), jnp.int32),
            pltpu.VMEM((CHUNK2, 16), jnp.float32),
            pltpu.VMEM((CHUNK2, 16), jnp.float32),
            pltpu.VMEM((NP,), jnp.float32),
            pltpu.VMEM((NP,), jnp.float32),
            pltpu.VMEM_SHARED((NP, 16), jnp.float32),
            pltpu.SemaphoreType.DMA,
            pltpu.SemaphoreType.DMA,
            pltpu.SemaphoreType.DMA,
            pltpu.SemaphoreType.DMA,
        ],
    )(src, dst, T2, as2f, ad2f)


def _stage_c_body(p0_ref, p1_ref, t2_ref, a2f_ref, b2_ref, out_ref):
    acc2 = p0_ref[...] + p1_ref[...]
    a2f = a2f_ref[...]
    ws2 = _lrelu_exp(a2f[:, 0:1] + a2f[:, 1:2])
    numf = acc2 + ws2 * t2_ref[...]
    den2 = numf[:, 10:11]
    logits = numf / den2 + b2_ref[...]
    lane = lax.broadcasted_iota(jnp.int32, (1, 16), 1)
    mask = lane < 10
    lm = jnp.where(mask, logits, -1e30)
    m = jnp.max(lm, axis=1, keepdims=True)
    ex = jnp.where(mask, jnp.exp(lm - m), 0.0)
    out_ref[...] = lm - m - jnp.log(jnp.sum(ex, axis=1, keepdims=True))


def _stage_c(P20, P21, T2, A2f, b2m):
    mb = 2048
    grid = (NP // mb,)
    return pl.pallas_call(
        _stage_c_body,
        grid=grid,
        in_specs=[
            pl.BlockSpec((mb, 16), lambda i: (i, 0)),
            pl.BlockSpec((mb, 16), lambda i: (i, 0)),
            pl.BlockSpec((mb, 16), lambda i: (i, 0)),
            pl.BlockSpec((mb, 16), lambda i: (i, 0)),
            pl.BlockSpec((1, 16), lambda i: (0, 0)),
        ],
        out_specs=pl.BlockSpec((mb, 16), lambda i: (i, 0)),
        out_shape=jax.ShapeDtypeStruct((NP, 16), jnp.float32),
    )(P20, P21, T2, A2f, b2m)


def kernel(x, edge_index, W1, a1_src, a1_dst, b1, W2, a2_src, a2_dst, b2):
    pad = jnp.full((EP - E,), N, jnp.int32)
    src = jnp.concatenate([edge_index[0], pad])
    dst = jnp.concatenate([edge_index[1], pad])
    xp = jnp.pad(x, ((0, NP - N), (0, 0)))
    eye8 = jnp.eye(8, dtype=jnp.float32)
    A1s = jnp.pad((a1_src[:, :, None] * eye8[:, None, :]).reshape(D1, 8),
                  ((0, 0), (0, 8)))
    A1d = jnp.pad((a1_dst[:, :, None] * eye8[:, None, :]).reshape(D1, 8),
                  ((0, 0), (0, 8)))
    Rsel = jnp.repeat(jnp.eye(16, dtype=jnp.float32)[:, :8], 16, axis=1)
    W2p = jnp.pad(W2, ((0, 0), (0, 6)))
    A2W = jnp.concatenate(
        [jnp.pad(a2_src[0], (0, 6))[:, None], jnp.pad(a2_dst[0], (0, 6))[:, None],
         jnp.zeros((16, 14), jnp.float32)], axis=1)
    b1m = b1[None, :]
    b2m = jnp.pad(b2, (0, 6))[None, :]

    HX, AD = _stage_a(xp, W1, A1s, A1d)
    P = _sc1(src, dst, HX, AD)
    T2, A2f = _stage_b(P[0], P[1], HX, AD, b1m, W2p, A2W, Rsel)
    P2 = _sc2(src, dst, T2, A2f[:, 0], A2f[:, 1])
    C = _stage_c(P2[0], P2[1], T2, A2f, b2m)
    return C[:N, :10]


# speedup vs baseline: 1.1297x; 1.1297x over previous
import jax
import jax.numpy as jnp
from jax import lax
from jax.experimental import pallas as pl
from jax.experimental.pallas import tpu as pltpu
from jax.experimental.pallas import tpu_sc as plsc

N = 10000
NP = 10240
E = 320000
EP = 327680
FIN = 128
D1 = 128
ACC_W = 144
NCORES = 2
NSUB = 16
NTILES = NCORES * NSUB
EDGES_PER_TILE = EP // NTILES
CHUNK1 = 80
EPT_C0 = 13440
EPT_C1 = 7040
NC0 = EPT_C0 // CHUNK1
NC1 = EPT_C1 // CHUNK1
CHUNK2 = 128
NCHUNK2 = EDGES_PER_TILE // CHUNK2
ROWS_PER_TILE = NP // NSUB

_HI = jax.lax.Precision.HIGHEST


def _lrelu_exp(v):
    return jnp.exp(jnp.maximum(v, 0.0) + 0.2 * jnp.minimum(v, 0.0))


def _lane_bcast(v, lane):
    idx = jnp.full((16, 1), lane, jnp.int32)
    dn = lax.GatherDimensionNumbers(
        offset_dims=(), collapsed_slice_dims=(0,), start_index_map=(0,))
    return lax.gather(v, idx, dn, (1,),
                      mode=lax.GatherScatterMode.PROMISE_IN_BOUNDS)


def _stage_a_body(x_ref, w1_ref, a1s_ref, a1d_ref, hx_ref, ad_ref):
    h = lax.dot_general(x_ref[...], w1_ref[...], (((1,), (0,)), ((), ())),
                        precision=_HI, preferred_element_type=jnp.float32)
    hx_ref[:, :D1] = h
    hx_ref[:, D1:ACC_W] = lax.dot_general(
        h, a1s_ref[...], (((1,), (0,)), ((), ())),
        precision=_HI, preferred_element_type=jnp.float32)
    ad_ref[...] = lax.dot_general(h, a1d_ref[...], (((1,), (0,)), ((), ())),
                                  precision=_HI,
                                  preferred_element_type=jnp.float32)


def _stage_a(xp, W1, A1s, A1d):
    mb = 2048
    grid = (NP // mb,)
    return pl.pallas_call(
        _stage_a_body,
        grid=grid,
        in_specs=[
            pl.BlockSpec((mb, FIN), lambda i: (i, 0)),
            pl.BlockSpec((FIN, D1), lambda i: (0, 0)),
            pl.BlockSpec((D1, 16), lambda i: (0, 0)),
            pl.BlockSpec((D1, 16), lambda i: (0, 0)),
        ],
        out_specs=[
            pl.BlockSpec((mb, ACC_W), lambda i: (i, 0)),
            pl.BlockSpec((mb, 16), lambda i: (i, 0)),
        ],
        out_shape=[
            jax.ShapeDtypeStruct((NP, ACC_W), jnp.float32),
            jax.ShapeDtypeStruct((NP, 16), jnp.float32),
        ],
    )(xp, W1, A1s, A1d)


def _sc1_body(src_hbm, dst_hbm, hx_hbm, ad_hbm, out_hbm,
              sidxA, didxA, sidxB, didxB, adrA, adrB, msgA, msgB,
              acc, gA, gB, sA, sB):
    c = lax.axis_index("c")
    s = lax.axis_index("s")
    wid = c * NSUB + s
    zero16 = jnp.zeros((16,), jnp.float32)

    @pl.loop(0, CHUNK1)
    def _zrow(i):
        @pl.loop(0, ACC_W // 16)
        def _zcol(j):
            msgA[i, pl.ds(j * 16, 16)] = zero16

    @pl.loop(0, ROWS_PER_TILE // CHUNK1)
    def _zacc(i):
        pltpu.sync_copy(msgA, acc.at[pl.ds(s * ROWS_PER_TILE + i * CHUNK1,
                                           CHUNK1)])

    plsc.subcore_barrier()

    def pipeline(base_w, nchunk):
        def load_idx(i, sidx, didx):
            b = base_w + i * CHUNK1
            pltpu.sync_copy(src_hbm.at[pl.ds(b, CHUNK1)], sidx)
            pltpu.sync_copy(dst_hbm.at[pl.ds(b, CHUNK1)], didx)

        def start_gather(sidx, didx, adr, msg, sem):
            pltpu.async_copy(hx_hbm.at[sidx], msg, sem)
            pltpu.async_copy(ad_hbm.at[didx], adr, sem)

        def wait_gather(sidx, didx, adr, msg, sem):
            pltpu.make_async_copy(hx_hbm.at[sidx], msg, sem).wait()
            pltpu.make_async_copy(ad_hbm.at[didx], adr, sem).wait()

        def compute(adr, msg):
            @plsc.parallel_loop(0, CHUNK1, 1, unroll=2)
            def _edge(e):
                v = msg[e, pl.ds(D1, 16)] + adr[e, :]
                w = _lrelu_exp(v)
                msg[e, pl.ds(D1, 16)] = w
                for hh in range(8):
                    bc = _lane_bcast(w, hh)
                    msg[e, pl.ds(hh * 16, 16)] = msg[e, pl.ds(hh * 16, 16)] * bc

        def start_scatter(msg, didx, sem):
            pltpu.async_copy(msg, acc.at[didx], sem, add=True)

        def wait_scatter(msg, didx, sem):
            pltpu.make_async_copy(msg, acc.at[didx], sem).wait()

        load_idx(0, sidxA, didxA)
        start_gather(sidxA, didxA, adrA, msgA, gA)
        load_idx(1, sidxB, didxB)
        start_gather(sidxB, didxB, adrB, msgB, gB)

        @pl.loop(0, nchunk // 2 - 1)
        def _round(r):
            i = 2 * r
            wait_gather(sidxA, didxA, adrA, msgA, gA)
            compute(adrA, msgA)
            start_scatter(msgA, didxA, sA)
            wait_gather(sidxB, didxB, adrB, msgB, gB)
            compute(adrB, msgB)
            start_scatter(msgB, didxB, sB)
            wait_scatter(msgA, didxA, sA)
            load_idx(i + 2, sidxA, didxA)
            start_gather(sidxA, didxA, adrA, msgA, gA)
            wait_scatter(msgB, didxB, sB)
            load_idx(i + 3, sidxB, didxB)
            start_gather(sidxB, didxB, adrB, msgB, gB)

        wait_gather(sidxA, didxA, adrA, msgA, gA)
        compute(adrA, msgA)
        start_scatter(msgA, didxA, sA)
        wait_gather(sidxB, didxB, adrB, msgB, gB)
        compute(adrB, msgB)
        start_scatter(msgB, didxB, sB)
        wait_scatter(msgA, didxA, sA)
        wait_scatter(msgB, didxB, sB)

    @pl.when(c == 0)
    def _p0():
        pipeline(s * EPT_C0, NC0)

    @pl.when(c == 1)
    def _p1():
        pipeline(EPT_C0 * NSUB + s * EPT_C1, NC1)

    plsc.subcore_barrier()
    row0 = s * ROWS_PER_TILE
    src_slice = acc.at[pl.ds(row0, ROWS_PER_TILE)]

    @pl.when(c == 0)
    def _w0():
        pltpu.sync_copy(src_slice, out_hbm.at[0, pl.ds(row0, ROWS_PER_TILE)])

    @pl.when(c == 1)
    def _w1():
        pltpu.sync_copy(src_slice, out_hbm.at[1, pl.ds(row0, ROWS_PER_TILE)])


def _sc1(src, dst, HX, AD):
    mesh = plsc.VectorSubcoreMesh(core_axis_name="c", subcore_axis_name="s")
    return pl.kernel(
        _sc1_body,
        mesh=mesh,
        compiler_params=pltpu.CompilerParams(use_tc_tiling_on_sc=False, needs_layout_passes=False),
        out_type=jax.ShapeDtypeStruct((2, NP, ACC_W), jnp.float32),
        scratch_types=[
            pltpu.VMEM((CHUNK1,), jnp.int32),
            pltpu.VMEM((CHUNK1,), jnp.int32),
            pltpu.VMEM((CHUNK1,), jnp.int32),
            pltpu.VMEM((CHUNK1,), jnp.int32),
            pltpu.VMEM((CHUNK1, 16), jnp.float32),
            pltpu.VMEM((CHUNK1, 16), jnp.float32),
            pltpu.VMEM((CHUNK1, ACC_W), jnp.float32),
            pltpu.VMEM((CHUNK1, ACC_W), jnp.float32),
            pltpu.VMEM_SHARED((NP, ACC_W), jnp.float32),
            pltpu.SemaphoreType.DMA,
            pltpu.SemaphoreType.DMA,
            pltpu.SemaphoreType.DMA,
            pltpu.SemaphoreType.DMA,
        ],
    )(src, dst, HX, AD)


def _stage_b_body(p0_ref, p1_ref, hx_ref, ad_ref, b1_ref, w2_ref,
                  a2w_ref, rsel_ref, t2_ref, a2f_ref):
    accf = p0_ref[...] + p1_ref[...]
    hx = hx_ref[...]
    ws = _lrelu_exp(hx[:, D1:ACC_W] + ad_ref[...])
    rsel = rsel_ref[...]
    ws_wide = lax.dot_general(ws, rsel, (((1,), (0,)), ((), ())),
                              precision=_HI, preferred_element_type=jnp.float32)
    den_wide = lax.dot_general(accf[:, D1:ACC_W] + ws, rsel,
                               (((1,), (0,)), ((), ())),
                               precision=_HI, preferred_element_type=jnp.float32)
    num = accf[:, :D1] + ws_wide * hx[:, :D1]
    out1 = num / den_wide + b1_ref[...]
    z = jnp.where(out1 > 0, out1, jnp.exp(jnp.minimum(out1, 0.0)) - 1.0)
    h2 = lax.dot_general(z, w2_ref[...], (((1,), (0,)), ((), ())),
                         precision=_HI, preferred_element_type=jnp.float32)
    lane = lax.broadcasted_iota(jnp.int32, (1, 16), 1)
    t2_ref[...] = h2 + jnp.where(lane == 10, 1.0, 0.0)
    a2f_ref[...] = lax.dot_general(h2, a2w_ref[...], (((1,), (0,)), ((), ())),
                                   precision=_HI,
                                   preferred_element_type=jnp.float32)


def _stage_b(P0, P1, HX, AD, b1m, W2p, A2W, Rsel):
    mb = 2048
    grid = (NP // mb,)
    return pl.pallas_call(
        _stage_b_body,
        grid=grid,
        in_specs=[
            pl.BlockSpec((mb, ACC_W), lambda i: (i, 0)),
            pl.BlockSpec((mb, ACC_W), lambda i: (i, 0)),
            pl.BlockSpec((mb, ACC_W), lambda i: (i, 0)),
            pl.BlockSpec((mb, 16), lambda i: (i, 0)),
            pl.BlockSpec((1, D1), lambda i: (0, 0)),
            pl.BlockSpec((D1, 16), lambda i: (0, 0)),
            pl.BlockSpec((16, 16), lambda i: (0, 0)),
            pl.BlockSpec((16, D1), lambda i: (0, 0)),
        ],
        out_specs=[
            pl.BlockSpec((mb, 16), lambda i: (i, 0)),
            pl.BlockSpec((mb, 16), lambda i: (i, 0)),
        ],
        out_shape=[
            jax.ShapeDtypeStruct((NP, 16), jnp.float32),
            jax.ShapeDtypeStruct((NP, 16), jnp.float32),
        ],
    )(P0, P1, HX, AD, b1m, W2p, A2W, Rsel)


def _sc2_body(src_hbm, dst_hbm, t2_hbm, as2_hbm, ad2_hbm, out_hbm,
              sidxA, didxA, sidxB, didxB, msgA, msgB, as2l, ad2l,
              acc, gA, gB, sA, sB):
    c = lax.axis_index("c")
    s = lax.axis_index("s")
    wid = c * NSUB + s
    pltpu.sync_copy(as2_hbm, as2l)
    pltpu.sync_copy(ad2_hbm, ad2l)
    zero16 = jnp.zeros((16,), jnp.float32)

    @pl.loop(0, CHUNK2)
    def _zrow(i):
        msgA[i, :] = zero16

    @pl.loop(0, ROWS_PER_TILE // CHUNK2)
    def _zacc(i):
        pltpu.sync_copy(msgA, acc.at[pl.ds(s * ROWS_PER_TILE + i * CHUNK2,
                                           CHUNK2)])

    plsc.subcore_barrier()

    base_w = wid * EDGES_PER_TILE

    def load_idx(i, sidx, didx):
        b = base_w + i * CHUNK2
        pltpu.sync_copy(src_hbm.at[pl.ds(b, CHUNK2)], sidx)
        pltpu.sync_copy(dst_hbm.at[pl.ds(b, CHUNK2)], didx)

    def start_gather(sidx, msg, sem):
        pltpu.async_copy(t2_hbm.at[sidx], msg, sem)

    def wait_gather(sidx, msg, sem):
        pltpu.make_async_copy(t2_hbm.at[sidx], msg, sem).wait()

    def compute(sidx, didx, msg):
        @plsc.parallel_loop(0, CHUNK2 // 16, 1, unroll=2)
        def _grp(g):
            sv = sidx[pl.ds(g * 16, 16)]
            dv = didx[pl.ds(g * 16, 16)]
            av = plsc.load_gather(as2l, [sv])
            bv = plsc.load_gather(ad2l, [dv])
            w2 = _lrelu_exp(av + bv)
            for j in range(16):
                bc = _lane_bcast(w2, j)
                msg[g * 16 + j, :] = msg[g * 16 + j, :] * bc

    def start_scatter(msg, didx, sem):
        pltpu.async_copy(msg, acc.at[didx], sem, add=True)

    def wait_scatter(msg, didx, sem):
        pltpu.make_async_copy(msg, acc.at[didx], sem).wait()

    load_idx(0, sidxA, didxA)
    start_gather(sidxA, msgA, gA)
    load_idx(1, sidxB, didxB)
    start_gather(sidxB, msgB, gB)

    @pl.loop(0, NCHUNK2 // 2 - 1)
    def _round(r):
        i = 2 * r
        wait_gather(sidxA, msgA, gA)
        compute(sidxA, didxA, msgA)
        start_scatter(msgA, didxA, sA)
        wait_gather(sidxB, msgB, gB)
        compute(sidxB, didxB, msgB)
        start_scatter(msgB, didxB, sB)
        wait_scatter(msgA, didxA, sA)
        load_idx(i + 2, sidxA, didxA)
        start_gather(sidxA, msgA, gA)
        wait_scatter(msgB, didxB, sB)
        load_idx(i + 3, sidxB, didxB)
        start_gather(sidxB, msgB, gB)

    wait_gather(sidxA, msgA, gA)
    compute(sidxA, didxA, msgA)
    start_scatter(msgA, didxA, sA)
    wait_gather(sidxB, msgB, gB)
    compute(sidxB, didxB, msgB)
    start_scatter(msgB, didxB, sB)
    wait_scatter(msgA, didxA, sA)
    wait_scatter(msgB, didxB, sB)

    plsc.subcore_barrier()
    row0 = s * ROWS_PER_TILE
    src_slice = acc.at[pl.ds(row0, ROWS_PER_TILE)]

    @pl.when(c == 0)
    def _w0():
        pltpu.sync_copy(src_slice, out_hbm.at[0, pl.ds(row0, ROWS_PER_TILE)])

    @pl.when(c == 1)
    def _w1():
        pltpu.sync_copy(src_slice, out_hbm.at[1, pl.ds(row0, ROWS_PER_TILE)])


def _sc2(src, dst, T2, as2f, ad2f):
    mesh = plsc.VectorSubcoreMesh(core_axis_name="c", subcore_axis_name="s")
    return pl.kernel(
        _sc2_body,
        mesh=mesh,
        compiler_params=pltpu.CompilerParams(use_tc_tiling_on_sc=False, needs_layout_passes=False),
        out_type=jax.ShapeDtypeStruct((2, NP, 16), jnp.float32),
        scratch_types=[
            pltpu.VMEM((CHUNK2,), jnp.int32),
            pltpu.VMEM((CHUNK2,), jnp.int32),
            pltpu.VMEM((CHUNK2,), jnp.int32),
            pltpu.VMEM((CHUNK2,), jnp.int32),
            pltpu.VMEM((CHUNK2, 16), jnp.float32),
            pltpu.VMEM((CHUNK2, 16), jnp.float32),
            pltpu.VMEM((NP,), jnp.float32),
            pltpu.VMEM((NP,), jnp.float32),
            pltpu.VMEM_SHARED((NP, 16), jnp.float32),
            pltpu.SemaphoreType.DMA,
            pltpu.SemaphoreType.DMA,
            pltpu.SemaphoreType.DMA,
            pltpu.SemaphoreType.DMA,
        ],
    )(src, dst, T2, as2f, ad2f)


def _stage_c_body(p0_ref, p1_ref, t2_ref, a2f_ref, b2_ref, out_ref):
    acc2 = p0_ref[...] + p1_ref[...]
    a2f = a2f_ref[...]
    ws2 = _lrelu_exp(a2f[:, 0:1] + a2f[:, 1:2])
    numf = acc2 + ws2 * t2_ref[...]
    den2 = numf[:, 10:11]
    logits = numf / den2 + b2_ref[...]
    lane = lax.broadcasted_iota(jnp.int32, (1, 16), 1)
    mask = lane < 10
    lm = jnp.where(mask, logits, -1e30)
    m = jnp.max(lm, axis=1, keepdims=True)
    ex = jnp.where(mask, jnp.exp(lm - m), 0.0)
    out_ref[...] = lm - m - jnp.log(jnp.sum(ex, axis=1, keepdims=True))


def _stage_c(P20, P21, T2, A2f, b2m):
    mb = 2048
    grid = (NP // mb,)
    return pl.pallas_call(
        _stage_c_body,
        grid=grid,
        in_specs=[
            pl.BlockSpec((mb, 16), lambda i: (i, 0)),
            pl.BlockSpec((mb, 16), lambda i: (i, 0)),
            pl.BlockSpec((mb, 16), lambda i: (i, 0)),
            pl.BlockSpec((mb, 16), lambda i: (i, 0)),
            pl.BlockSpec((1, 16), lambda i: (0, 0)),
        ],
        out_specs=pl.BlockSpec((mb, 16), lambda i: (i, 0)),
        out_shape=jax.ShapeDtypeStruct((NP, 16), jnp.float32),
    )(P20, P21, T2, A2f, b2m)


def kernel(x, edge_index, W1, a1_src, a1_dst, b1, W2, a2_src, a2_dst, b2):
    pad = jnp.full((EP - E,), N, jnp.int32)
    src = jnp.concatenate([edge_index[0], pad])
    dst = jnp.concatenate([edge_index[1], pad])
    xp = jnp.pad(x, ((0, NP - N), (0, 0)))
    eye8 = jnp.eye(8, dtype=jnp.float32)
    A1s = jnp.pad((a1_src[:, :, None] * eye8[:, None, :]).reshape(D1, 8),
                  ((0, 0), (0, 8)))
    A1d = jnp.pad((a1_dst[:, :, None] * eye8[:, None, :]).reshape(D1, 8),
                  ((0, 0), (0, 8)))
    Rsel = jnp.repeat(jnp.eye(16, dtype=jnp.float32)[:, :8], 16, axis=1)
    W2p = jnp.pad(W2, ((0, 0), (0, 6)))
    A2W = jnp.concatenate(
        [jnp.pad(a2_src[0], (0, 6))[:, None], jnp.pad(a2_dst[0], (0, 6))[:, None],
         jnp.zeros((16, 14), jnp.float32)], axis=1)
    b1m = b1[None, :]
    b2m = jnp.pad(b2, (0, 6))[None, :]

    HX, AD = _stage_a(xp, W1, A1s, A1d)
    P = _sc1(src, dst, HX, AD)
    T2, A2f = _stage_b(P[0], P[1], HX, AD, b1m, W2p, A2W, Rsel)
    P2 = _sc2(src, dst, T2, A2f[:, 0], A2f[:, 1])
    C = _stage_c(P2[0], P2[1], T2, A2f, b2m)
    return C[:N, :10]


# speedup vs baseline: 1.4731x; 1.3040x over previous
import jax
import jax.numpy as jnp
from jax import lax
from jax.experimental import pallas as pl
from jax.experimental.pallas import tpu as pltpu
from jax.experimental.pallas import tpu_sc as plsc

N = 10000
NP = 10240
E = 320000
EP = 322560
FIN = 128
D1 = 128
ACC_W = 144
NCORES = 2
NSUB = 16
NTILES = NCORES * NSUB
CHUNK1 = 80
EPT_C0 = 10080
EPT_C1 = 10080
NC0 = EPT_C0 // CHUNK1
NC1 = EPT_C1 // CHUNK1
EDGES_PER_TILE2 = EP // NTILES
CHUNK2 = 112
NCHUNK2 = EDGES_PER_TILE2 // CHUNK2
ROWS_PER_TILE = NP // NSUB

_HI = jax.lax.Precision.HIGHEST


def _lrelu_exp(v):
    return jnp.exp(jnp.maximum(v, 0.0) + 0.2 * jnp.minimum(v, 0.0))


def _lane_bcast(v, lane):
    idx = jnp.full((16, 1), lane, jnp.int32)
    dn = lax.GatherDimensionNumbers(
        offset_dims=(), collapsed_slice_dims=(0,), start_index_map=(0,))
    return lax.gather(v, idx, dn, (1,),
                      mode=lax.GatherScatterMode.PROMISE_IN_BOUNDS)


def _stage_a_body(x_ref, w1_ref, a1s_ref, a1d_ref, hx_ref, ad_ref):
    h = lax.dot_general(x_ref[...], w1_ref[...], (((1,), (0,)), ((), ())),
                        precision=_HI, preferred_element_type=jnp.float32)
    hx_ref[:, :D1] = h
    hx_ref[:, D1:ACC_W] = lax.dot_general(
        h, a1s_ref[...], (((1,), (0,)), ((), ())),
        precision=_HI, preferred_element_type=jnp.float32)
    ad_ref[...] = lax.dot_general(h, a1d_ref[...], (((1,), (0,)), ((), ())),
                                  precision=_HI,
                                  preferred_element_type=jnp.float32)


def _stage_a(xp, W1, A1s, A1d):
    mb = 2048
    grid = (NP // mb,)
    return pl.pallas_call(
        _stage_a_body,
        grid=grid,
        in_specs=[
            pl.BlockSpec((mb, FIN), lambda i: (i, 0)),
            pl.BlockSpec((FIN, D1), lambda i: (0, 0)),
            pl.BlockSpec((D1, 16), lambda i: (0, 0)),
            pl.BlockSpec((D1, 16), lambda i: (0, 0)),
        ],
        out_specs=[
            pl.BlockSpec((mb, ACC_W), lambda i: (i, 0)),
            pl.BlockSpec((mb, 16), lambda i: (i, 0)),
        ],
        out_shape=[
            jax.ShapeDtypeStruct((NP, ACC_W), jnp.float32),
            jax.ShapeDtypeStruct((NP, 16), jnp.float32),
        ],
    )(xp, W1, A1s, A1d)


def _sc1_body(src_hbm, dst_hbm, hx_hbm, ad_hbm, out_hbm,
              sidxA, didxA, sidxB, didxB, sidxC, didxC,
              adrA, adrB, adrC, msgA, msgB, msgC,
              acc, gA, gB, gC, sA, sB, sC):
    c = lax.axis_index("c")
    s = lax.axis_index("s")
    zero16 = jnp.zeros((16,), jnp.float32)

    @pl.loop(0, CHUNK1)
    def _zrow(i):
        @pl.loop(0, ACC_W // 16)
        def _zcol(j):
            msgA[i, pl.ds(j * 16, 16)] = zero16

    @pl.loop(0, ROWS_PER_TILE // CHUNK1)
    def _zacc(i):
        pltpu.sync_copy(msgA, acc.at[pl.ds(s * ROWS_PER_TILE + i * CHUNK1,
                                           CHUNK1)])

    plsc.subcore_barrier()

    sets = ((sidxA, didxA, adrA, msgA, gA, sA),
            (sidxB, didxB, adrB, msgB, gB, sB),
            (sidxC, didxC, adrC, msgC, gC, sC))

    def pipeline(base_w, nchunk):
        def load_and_gather(i, st):
            sidx, didx, adr, msg, g, _ = st
            b = base_w + i * CHUNK1
            pltpu.sync_copy(src_hbm.at[pl.ds(b, CHUNK1)], sidx)
            pltpu.sync_copy(dst_hbm.at[pl.ds(b, CHUNK1)], didx)
            pltpu.async_copy(hx_hbm.at[sidx], msg, g)
            pltpu.async_copy(ad_hbm.at[didx], adr, g)

        def do_chunk(st):
            sidx, didx, adr, msg, g, sc = st
            pltpu.make_async_copy(hx_hbm.at[sidx], msg, g).wait()
            pltpu.make_async_copy(ad_hbm.at[didx], adr, g).wait()

            @plsc.parallel_loop(0, CHUNK1, 1, unroll=2)
            def _edge(e):
                v = msg[e, pl.ds(D1, 16)] + adr[e, :]
                w = _lrelu_exp(v)
                msg[e, pl.ds(D1, 16)] = w
                for hh in range(8):
                    bc = _lane_bcast(w, hh)
                    msg[e, pl.ds(hh * 16, 16)] = (
                        msg[e, pl.ds(hh * 16, 16)] * bc)

            pltpu.async_copy(msg, acc.at[didx], sc, add=True)

        def wait_scatter(st):
            sidx, didx, adr, msg, g, sc = st
            pltpu.make_async_copy(msg, acc.at[didx], sc).wait()

        load_and_gather(0, sets[0])
        load_and_gather(1, sets[1])
        do_chunk(sets[0])
        load_and_gather(2, sets[2])

        @pl.loop(0, (nchunk - 3) // 3)
        def _round(r):
            i = 1 + 3 * r
            do_chunk(sets[1])
            wait_scatter(sets[0])
            load_and_gather(i + 2, sets[0])
            do_chunk(sets[2])
            wait_scatter(sets[1])
            load_and_gather(i + 3, sets[1])
            do_chunk(sets[0])
            wait_scatter(sets[2])
            load_and_gather(i + 4, sets[2])

        do_chunk(sets[1])
        do_chunk(sets[2])
        wait_scatter(sets[0])
        wait_scatter(sets[1])
        wait_scatter(sets[2])

    @pl.when(c == 0)
    def _p0():
        pipeline(s * EPT_C0, NC0)

    @pl.when(c == 1)
    def _p1():
        pipeline(EPT_C0 * NSUB + s * EPT_C1, NC1)

    plsc.subcore_barrier()
    row0 = s * ROWS_PER_TILE
    src_slice = acc.at[pl.ds(row0, ROWS_PER_TILE)]

    @pl.when(c == 0)
    def _w0():
        pltpu.sync_copy(src_slice, out_hbm.at[0, pl.ds(row0, ROWS_PER_TILE)])

    @pl.when(c == 1)
    def _w1():
        pltpu.sync_copy(src_slice, out_hbm.at[1, pl.ds(row0, ROWS_PER_TILE)])


def _sc1(src, dst, HX, AD):
    mesh = plsc.VectorSubcoreMesh(core_axis_name="c", subcore_axis_name="s")
    return pl.kernel(
        _sc1_body,
        mesh=mesh,
        compiler_params=pltpu.CompilerParams(use_tc_tiling_on_sc=False, needs_layout_passes=False),
        out_type=jax.ShapeDtypeStruct((2, NP, ACC_W), jnp.float32),
        scratch_types=[
            pltpu.VMEM((CHUNK1,), jnp.int32),
            pltpu.VMEM((CHUNK1,), jnp.int32),
            pltpu.VMEM((CHUNK1,), jnp.int32),
            pltpu.VMEM((CHUNK1,), jnp.int32),
            pltpu.VMEM((CHUNK1,), jnp.int32),
            pltpu.VMEM((CHUNK1,), jnp.int32),
            pltpu.VMEM((CHUNK1, 16), jnp.float32),
            pltpu.VMEM((CHUNK1, 16), jnp.float32),
            pltpu.VMEM((CHUNK1, 16), jnp.float32),
            pltpu.VMEM((CHUNK1, ACC_W), jnp.float32),
            pltpu.VMEM((CHUNK1, ACC_W), jnp.float32),
            pltpu.VMEM((CHUNK1, ACC_W), jnp.float32),
            pltpu.VMEM_SHARED((NP, ACC_W), jnp.float32),
            pltpu.SemaphoreType.DMA,
            pltpu.SemaphoreType.DMA,
            pltpu.SemaphoreType.DMA,
            pltpu.SemaphoreType.DMA,
            pltpu.SemaphoreType.DMA,
            pltpu.SemaphoreType.DMA,
        ],
    )(src, dst, HX, AD)


def _stage_b_body(p0_ref, p1_ref, hx_ref, ad_ref, b1_ref, w2_ref,
                  a2w_ref, rsel_ref, t2_ref, a2f_ref):
    accf = p0_ref[...] + p1_ref[...]
    hx = hx_ref[...]
    ws = _lrelu_exp(hx[:, D1:ACC_W] + ad_ref[...])
    rsel = rsel_ref[...]
    ws_wide = lax.dot_general(ws, rsel, (((1,), (0,)), ((), ())),
                              precision=_HI, preferred_element_type=jnp.float32)
    den_wide = lax.dot_general(accf[:, D1:ACC_W] + ws, rsel,
                               (((1,), (0,)), ((), ())),
                               precision=_HI, preferred_element_type=jnp.float32)
    num = accf[:, :D1] + ws_wide * hx[:, :D1]
    out1 = num / den_wide + b1_ref[...]
    z = jnp.where(out1 > 0, out1, jnp.exp(jnp.minimum(out1, 0.0)) - 1.0)
    h2 = lax.dot_general(z, w2_ref[...], (((1,), (0,)), ((), ())),
                         precision=_HI, preferred_element_type=jnp.float32)
    lane = lax.broadcasted_iota(jnp.int32, (1, 16), 1)
    t2_ref[...] = h2 + jnp.where(lane == 10, 1.0, 0.0)
    a2f_ref[...] = lax.dot_general(h2, a2w_ref[...], (((1,), (0,)), ((), ())),
                                   precision=_HI,
                                   preferred_element_type=jnp.float32)


def _stage_b(P0, P1, HX, AD, b1m, W2p, A2W, Rsel):
    mb = 2048
    grid = (NP // mb,)
    return pl.pallas_call(
        _stage_b_body,
        grid=grid,
        in_specs=[
            pl.BlockSpec((mb, ACC_W), lambda i: (i, 0)),
            pl.BlockSpec((mb, ACC_W), lambda i: (i, 0)),
            pl.BlockSpec((mb, ACC_W), lambda i: (i, 0)),
            pl.BlockSpec((mb, 16), lambda i: (i, 0)),
            pl.BlockSpec((1, D1), lambda i: (0, 0)),
            pl.BlockSpec((D1, 16), lambda i: (0, 0)),
            pl.BlockSpec((16, 16), lambda i: (0, 0)),
            pl.BlockSpec((16, D1), lambda i: (0, 0)),
        ],
        out_specs=[
            pl.BlockSpec((mb, 16), lambda i: (i, 0)),
            pl.BlockSpec((mb, 16), lambda i: (i, 0)),
        ],
        out_shape=[
            jax.ShapeDtypeStruct((NP, 16), jnp.float32),
            jax.ShapeDtypeStruct((NP, 16), jnp.float32),
        ],
    )(P0, P1, HX, AD, b1m, W2p, A2W, Rsel)


def _sc2_body(src_hbm, dst_hbm, t2_hbm, as2_hbm, ad2_hbm, out_hbm,
              sidxA, didxA, sidxB, didxB, msgA, msgB, as2l, ad2l,
              acc, gA, gB, sA, sB):
    c = lax.axis_index("c")
    s = lax.axis_index("s")
    wid = c * NSUB + s
    pltpu.sync_copy(as2_hbm, as2l)
    pltpu.sync_copy(ad2_hbm, ad2l)
    zero16 = jnp.zeros((16,), jnp.float32)

    @pl.loop(0, CHUNK2)
    def _zrow(i):
        msgA[i, :] = zero16

    @pl.loop(0, ROWS_PER_TILE // 64)
    def _zacc(i):
        pltpu.sync_copy(msgA.at[pl.ds(0, 64)],
                        acc.at[pl.ds(s * ROWS_PER_TILE + i * 64, 64)])

    plsc.subcore_barrier()

    base_w = wid * EDGES_PER_TILE2

    def load_idx(i, sidx, didx):
        b = base_w + i * CHUNK2
        pltpu.sync_copy(src_hbm.at[pl.ds(b, CHUNK2)], sidx)
        pltpu.sync_copy(dst_hbm.at[pl.ds(b, CHUNK2)], didx)

    def start_gather(sidx, msg, sem):
        pltpu.async_copy(t2_hbm.at[sidx], msg, sem)

    def wait_gather(sidx, msg, sem):
        pltpu.make_async_copy(t2_hbm.at[sidx], msg, sem).wait()

    def compute(sidx, didx, msg):
        @plsc.parallel_loop(0, CHUNK2 // 16, 1, unroll=2)
        def _grp(g):
            sv = sidx[pl.ds(g * 16, 16)]
            dv = didx[pl.ds(g * 16, 16)]
            av = plsc.load_gather(as2l, [sv])
            bv = plsc.load_gather(ad2l, [dv])
            w2 = _lrelu_exp(av + bv)
            for j in range(16):
                bc = _lane_bcast(w2, j)
                msg[g * 16 + j, :] = msg[g * 16 + j, :] * bc

    def start_scatter(msg, didx, sem):
        pltpu.async_copy(msg, acc.at[didx], sem, add=True)

    def wait_scatter(msg, didx, sem):
        pltpu.make_async_copy(msg, acc.at[didx], sem).wait()

    load_idx(0, sidxA, didxA)
    start_gather(sidxA, msgA, gA)
    load_idx(1, sidxB, didxB)
    start_gather(sidxB, msgB, gB)

    @pl.loop(0, NCHUNK2 // 2 - 1)
    def _round(r):
        i = 2 * r
        wait_gather(sidxA, msgA, gA)
        compute(sidxA, didxA, msgA)
        start_scatter(msgA, didxA, sA)
        wait_gather(sidxB, msgB, gB)
        compute(sidxB, didxB, msgB)
        start_scatter(msgB, didxB, sB)
        wait_scatter(msgA, didxA, sA)
        load_idx(i + 2, sidxA, didxA)
        start_gather(sidxA, msgA, gA)
        wait_scatter(msgB, didxB, sB)
        load_idx(i + 3, sidxB, didxB)
        start_gather(sidxB, msgB, gB)

    wait_gather(sidxA, msgA, gA)
    compute(sidxA, didxA, msgA)
    start_scatter(msgA, didxA, sA)
    wait_gather(sidxB, msgB, gB)
    compute(sidxB, didxB, msgB)
    start_scatter(msgB, didxB, sB)
    wait_scatter(msgA, didxA, sA)
    wait_scatter(msgB, didxB, sB)

    plsc.subcore_barrier()
    row0 = s * ROWS_PER_TILE
    src_slice = acc.at[pl.ds(row0, ROWS_PER_TILE)]

    @pl.when(c == 0)
    def _w0():
        pltpu.sync_copy(src_slice, out_hbm.at[0, pl.ds(row0, ROWS_PER_TILE)])

    @pl.when(c == 1)
    def _w1():
        pltpu.sync_copy(src_slice, out_hbm.at[1, pl.ds(row0, ROWS_PER_TILE)])


def _sc2(src, dst, T2, as2f, ad2f):
    mesh = plsc.VectorSubcoreMesh(core_axis_name="c", subcore_axis_name="s")
    return pl.kernel(
        _sc2_body,
        mesh=mesh,
        compiler_params=pltpu.CompilerParams(use_tc_tiling_on_sc=False, needs_layout_passes=False),
        out_type=jax.ShapeDtypeStruct((2, NP, 16), jnp.float32),
        scratch_types=[
            pltpu.VMEM((CHUNK2,), jnp.int32),
            pltpu.VMEM((CHUNK2,), jnp.int32),
            pltpu.VMEM((CHUNK2,), jnp.int32),
            pltpu.VMEM((CHUNK2,), jnp.int32),
            pltpu.VMEM((CHUNK2, 16), jnp.float32),
            pltpu.VMEM((CHUNK2, 16), jnp.float32),
            pltpu.VMEM((NP,), jnp.float32),
            pltpu.VMEM((NP,), jnp.float32),
            pltpu.VMEM_SHARED((NP, 16), jnp.float32),
            pltpu.SemaphoreType.DMA,
            pltpu.SemaphoreType.DMA,
            pltpu.SemaphoreType.DMA,
            pltpu.SemaphoreType.DMA,
        ],
    )(src, dst, T2, as2f, ad2f)


def _stage_c_body(p0_ref, p1_ref, t2_ref, a2f_ref, b2_ref, out_ref):
    acc2 = p0_ref[...] + p1_ref[...]
    a2f = a2f_ref[...]
    ws2 = _lrelu_exp(a2f[:, 0:1] + a2f[:, 1:2])
    numf = acc2 + ws2 * t2_ref[...]
    den2 = numf[:, 10:11]
    logits = numf / den2 + b2_ref[...]
    lane = lax.broadcasted_iota(jnp.int32, (1, 16), 1)
    mask = lane < 10
    lm = jnp.where(mask, logits, -1e30)
    m = jnp.max(lm, axis=1, keepdims=True)
    ex = jnp.where(mask, jnp.exp(lm - m), 0.0)
    out_ref[...] = lm - m - jnp.log(jnp.sum(ex, axis=1, keepdims=True))


def _stage_c(P20, P21, T2, A2f, b2m):
    mb = 2048
    grid = (NP // mb,)
    return pl.pallas_call(
        _stage_c_body,
        grid=grid,
        in_specs=[
            pl.BlockSpec((mb, 16), lambda i: (i, 0)),
            pl.BlockSpec((mb, 16), lambda i: (i, 0)),
            pl.BlockSpec((mb, 16), lambda i: (i, 0)),
            pl.BlockSpec((mb, 16), lambda i: (i, 0)),
            pl.BlockSpec((1, 16), lambda i: (0, 0)),
        ],
        out_specs=pl.BlockSpec((mb, 16), lambda i: (i, 0)),
        out_shape=jax.ShapeDtypeStruct((NP, 16), jnp.float32),
    )(P20, P21, T2, A2f, b2m)


def kernel(x, edge_index, W1, a1_src, a1_dst, b1, W2, a2_src, a2_dst, b2):
    pad = jnp.full((EP - E,), N, jnp.int32)
    src = jnp.concatenate([edge_index[0], pad])
    dst = jnp.concatenate([edge_index[1], pad])
    xp = jnp.pad(x, ((0, NP - N), (0, 0)))
    eye8 = jnp.eye(8, dtype=jnp.float32)
    A1s = jnp.pad((a1_src[:, :, None] * eye8[:, None, :]).reshape(D1, 8),
                  ((0, 0), (0, 8)))
    A1d = jnp.pad((a1_dst[:, :, None] * eye8[:, None, :]).reshape(D1, 8),
                  ((0, 0), (0, 8)))
    Rsel = jnp.repeat(jnp.eye(16, dtype=jnp.float32)[:, :8], 16, axis=1)
    W2p = jnp.pad(W2, ((0, 0), (0, 6)))
    A2W = jnp.concatenate(
        [jnp.pad(a2_src[0], (0, 6))[:, None], jnp.pad(a2_dst[0], (0, 6))[:, None],
         jnp.zeros((16, 14), jnp.float32)], axis=1)
    b1m = b1[None, :]
    b2m = jnp.pad(b2, (0, 6))[None, :]

    HX, AD = _stage_a(xp, W1, A1s, A1d)
    P = _sc1(src, dst, HX, AD)
    T2, A2f = _stage_b(P[0], P[1], HX, AD, b1m, W2p, A2W, Rsel)
    P2 = _sc2(src, dst, T2, A2f[:, 0], A2f[:, 1])
    C = _stage_c(P2[0], P2[1], T2, A2f, b2m)
    return C[:N, :10]


# speedup vs baseline: 1.5681x; 1.0644x over previous
import jax
import jax.numpy as jnp
from jax import lax
from jax.experimental import pallas as pl
from jax.experimental.pallas import tpu as pltpu
from jax.experimental.pallas import tpu_sc as plsc

N = 10000
NP = 10240
E = 320000
EP = 322560
FIN = 128
D1 = 128
ACC_W = 144
NCORES = 2
NSUB = 16
NTILES = NCORES * NSUB
CHUNK1 = 80
EPT_C0 = 11520
EPT_C1 = 8640
NC0 = EPT_C0 // CHUNK1
NC1 = EPT_C1 // CHUNK1
EDGES_PER_TILE2 = EP // NTILES
CHUNK2 = 112
NCHUNK2 = EDGES_PER_TILE2 // CHUNK2
ROWS_PER_TILE = NP // NSUB

_HI = jax.lax.Precision.HIGHEST


def _lrelu_exp(v):
    return jnp.exp(jnp.maximum(v, 0.0) + 0.2 * jnp.minimum(v, 0.0))


def _lane_bcast(v, lane):
    idx = jnp.full((16, 1), lane, jnp.int32)
    dn = lax.GatherDimensionNumbers(
        offset_dims=(), collapsed_slice_dims=(0,), start_index_map=(0,))
    return lax.gather(v, idx, dn, (1,),
                      mode=lax.GatherScatterMode.PROMISE_IN_BOUNDS)


def _stage_a_body(x_ref, w1_ref, a1s_ref, a1d_ref, hx_ref, ad_ref):
    h = lax.dot_general(x_ref[...], w1_ref[...], (((1,), (0,)), ((), ())),
                        precision=_HI, preferred_element_type=jnp.float32)
    hx_ref[:, :D1] = h
    hx_ref[:, D1:ACC_W] = lax.dot_general(
        h, a1s_ref[...], (((1,), (0,)), ((), ())),
        precision=_HI, preferred_element_type=jnp.float32)
    ad_ref[...] = lax.dot_general(h, a1d_ref[...], (((1,), (0,)), ((), ())),
                                  precision=_HI,
                                  preferred_element_type=jnp.float32)


def _stage_a(xp, W1, A1s, A1d):
    mb = 2048
    grid = (NP // mb,)
    return pl.pallas_call(
        _stage_a_body,
        grid=grid,
        in_specs=[
            pl.BlockSpec((mb, FIN), lambda i: (i, 0)),
            pl.BlockSpec((FIN, D1), lambda i: (0, 0)),
            pl.BlockSpec((D1, 16), lambda i: (0, 0)),
            pl.BlockSpec((D1, 16), lambda i: (0, 0)),
        ],
        out_specs=[
            pl.BlockSpec((mb, ACC_W), lambda i: (i, 0)),
            pl.BlockSpec((mb, 16), lambda i: (i, 0)),
        ],
        out_shape=[
            jax.ShapeDtypeStruct((NP, ACC_W), jnp.float32),
            jax.ShapeDtypeStruct((NP, 16), jnp.float32),
        ],
    )(xp, W1, A1s, A1d)


def _sc1_body(src_hbm, dst_hbm, hx_hbm, ad_hbm, out_hbm,
              sidxA, didxA, sidxB, didxB, sidxC, didxC,
              adrA, adrB, adrC, msgA, msgB, msgC,
              acc, gA, gB, gC, sA, sB, sC):
    c = lax.axis_index("c")
    s = lax.axis_index("s")
    zero16 = jnp.zeros((16,), jnp.float32)

    @pl.loop(0, CHUNK1)
    def _zrow(i):
        @pl.loop(0, ACC_W // 16)
        def _zcol(j):
            msgA[i, pl.ds(j * 16, 16)] = zero16

    @pl.loop(0, ROWS_PER_TILE // CHUNK1)
    def _zacc(i):
        pltpu.sync_copy(msgA, acc.at[pl.ds(s * ROWS_PER_TILE + i * CHUNK1,
                                           CHUNK1)])

    plsc.subcore_barrier()

    sets = ((sidxA, didxA, adrA, msgA, gA, sA),
            (sidxB, didxB, adrB, msgB, gB, sB),
            (sidxC, didxC, adrC, msgC, gC, sC))

    def pipeline(base_w, nchunk):
        def load_and_gather(i, st):
            sidx, didx, adr, msg, g, _ = st
            b = base_w + i * CHUNK1
            pltpu.sync_copy(src_hbm.at[pl.ds(b, CHUNK1)], sidx)
            pltpu.sync_copy(dst_hbm.at[pl.ds(b, CHUNK1)], didx)
            pltpu.async_copy(hx_hbm.at[sidx], msg, g)
            pltpu.async_copy(ad_hbm.at[didx], adr, g)

        def do_chunk(st):
            sidx, didx, adr, msg, g, sc = st
            pltpu.make_async_copy(hx_hbm.at[sidx], msg, g).wait()
            pltpu.make_async_copy(ad_hbm.at[didx], adr, g).wait()

            @plsc.parallel_loop(0, CHUNK1, 1, unroll=2)
            def _edge(e):
                v = msg[e, pl.ds(D1, 16)] + adr[e, :]
                w = _lrelu_exp(v)
                msg[e, pl.ds(D1, 16)] = w
                for hh in range(8):
                    bc = _lane_bcast(w, hh)
                    msg[e, pl.ds(hh * 16, 16)] = (
                        msg[e, pl.ds(hh * 16, 16)] * bc)

            pltpu.async_copy(msg, acc.at[didx], sc, add=True)

        def wait_scatter(st):
            sidx, didx, adr, msg, g, sc = st
            pltpu.make_async_copy(msg, acc.at[didx], sc).wait()

        load_and_gather(0, sets[0])
        load_and_gather(1, sets[1])
        do_chunk(sets[0])
        load_and_gather(2, sets[2])

        @pl.loop(0, (nchunk - 3) // 3)
        def _round(r):
            i = 1 + 3 * r
            do_chunk(sets[1])
            wait_scatter(sets[0])
            load_and_gather(i + 2, sets[0])
            do_chunk(sets[2])
            wait_scatter(sets[1])
            load_and_gather(i + 3, sets[1])
            do_chunk(sets[0])
            wait_scatter(sets[2])
            load_and_gather(i + 4, sets[2])

        do_chunk(sets[1])
        do_chunk(sets[2])
        wait_scatter(sets[0])
        wait_scatter(sets[1])
        wait_scatter(sets[2])

    @pl.when(c == 0)
    def _p0():
        pipeline(s * EPT_C0, NC0)

    @pl.when(c == 1)
    def _p1():
        pipeline(EPT_C0 * NSUB + s * EPT_C1, NC1)

    plsc.subcore_barrier()
    row0 = s * ROWS_PER_TILE
    src_slice = acc.at[pl.ds(row0, ROWS_PER_TILE)]

    @pl.when(c == 0)
    def _w0():
        pltpu.sync_copy(src_slice, out_hbm.at[0, pl.ds(row0, ROWS_PER_TILE)])

    @pl.when(c == 1)
    def _w1():
        pltpu.sync_copy(src_slice, out_hbm.at[1, pl.ds(row0, ROWS_PER_TILE)])


def _sc1(src, dst, HX, AD):
    mesh = plsc.VectorSubcoreMesh(core_axis_name="c", subcore_axis_name="s")
    return pl.kernel(
        _sc1_body,
        mesh=mesh,
        compiler_params=pltpu.CompilerParams(use_tc_tiling_on_sc=False, needs_layout_passes=False),
        out_type=jax.ShapeDtypeStruct((2, NP, ACC_W), jnp.float32),
        scratch_types=[
            pltpu.VMEM((CHUNK1,), jnp.int32),
            pltpu.VMEM((CHUNK1,), jnp.int32),
            pltpu.VMEM((CHUNK1,), jnp.int32),
            pltpu.VMEM((CHUNK1,), jnp.int32),
            pltpu.VMEM((CHUNK1,), jnp.int32),
            pltpu.VMEM((CHUNK1,), jnp.int32),
            pltpu.VMEM((CHUNK1, 16), jnp.float32),
            pltpu.VMEM((CHUNK1, 16), jnp.float32),
            pltpu.VMEM((CHUNK1, 16), jnp.float32),
            pltpu.VMEM((CHUNK1, ACC_W), jnp.float32),
            pltpu.VMEM((CHUNK1, ACC_W), jnp.float32),
            pltpu.VMEM((CHUNK1, ACC_W), jnp.float32),
            pltpu.VMEM_SHARED((NP, ACC_W), jnp.float32),
            pltpu.SemaphoreType.DMA,
            pltpu.SemaphoreType.DMA,
            pltpu.SemaphoreType.DMA,
            pltpu.SemaphoreType.DMA,
            pltpu.SemaphoreType.DMA,
            pltpu.SemaphoreType.DMA,
        ],
    )(src, dst, HX, AD)


def _stage_b_body(p0_ref, p1_ref, hx_ref, ad_ref, b1_ref, w2_ref,
                  a2w_ref, rsel_ref, t2_ref, a2f_ref):
    accf = p0_ref[...] + p1_ref[...]
    hx = hx_ref[...]
    ws = _lrelu_exp(hx[:, D1:ACC_W] + ad_ref[...])
    rsel = rsel_ref[...]
    ws_wide = lax.dot_general(ws, rsel, (((1,), (0,)), ((), ())),
                              precision=_HI, preferred_element_type=jnp.float32)
    den_wide = lax.dot_general(accf[:, D1:ACC_W] + ws, rsel,
                               (((1,), (0,)), ((), ())),
                               precision=_HI, preferred_element_type=jnp.float32)
    num = accf[:, :D1] + ws_wide * hx[:, :D1]
    out1 = num / den_wide + b1_ref[...]
    z = jnp.where(out1 > 0, out1, jnp.exp(jnp.minimum(out1, 0.0)) - 1.0)
    h2 = lax.dot_general(z, w2_ref[...], (((1,), (0,)), ((), ())),
                         precision=_HI, preferred_element_type=jnp.float32)
    lane = lax.broadcasted_iota(jnp.int32, (1, 16), 1)
    t2_ref[...] = h2 + jnp.where(lane == 10, 1.0, 0.0)
    a2f_ref[...] = lax.dot_general(h2, a2w_ref[...], (((1,), (0,)), ((), ())),
                                   precision=_HI,
                                   preferred_element_type=jnp.float32)


def _stage_b(P0, P1, HX, AD, b1m, W2p, A2W, Rsel):
    mb = 2048
    grid = (NP // mb,)
    return pl.pallas_call(
        _stage_b_body,
        grid=grid,
        in_specs=[
            pl.BlockSpec((mb, ACC_W), lambda i: (i, 0)),
            pl.BlockSpec((mb, ACC_W), lambda i: (i, 0)),
            pl.BlockSpec((mb, ACC_W), lambda i: (i, 0)),
            pl.BlockSpec((mb, 16), lambda i: (i, 0)),
            pl.BlockSpec((1, D1), lambda i: (0, 0)),
            pl.BlockSpec((D1, 16), lambda i: (0, 0)),
            pl.BlockSpec((16, 16), lambda i: (0, 0)),
            pl.BlockSpec((16, D1), lambda i: (0, 0)),
        ],
        out_specs=[
            pl.BlockSpec((mb, 16), lambda i: (i, 0)),
            pl.BlockSpec((mb, 16), lambda i: (i, 0)),
        ],
        out_shape=[
            jax.ShapeDtypeStruct((NP, 16), jnp.float32),
            jax.ShapeDtypeStruct((NP, 16), jnp.float32),
        ],
    )(P0, P1, HX, AD, b1m, W2p, A2W, Rsel)


def _sc2_body(src_hbm, dst_hbm, t2_hbm, as2_hbm, ad2_hbm, out_hbm,
              sidxA, didxA, sidxB, didxB, sidxC, didxC,
              msgA, msgB, msgC, as2l, ad2l,
              acc, gA, gB, gC, sA, sB, sC):
    c = lax.axis_index("c")
    s = lax.axis_index("s")
    wid = c * NSUB + s
    pltpu.sync_copy(as2_hbm, as2l)
    pltpu.sync_copy(ad2_hbm, ad2l)
    zero16 = jnp.zeros((16,), jnp.float32)

    @pl.loop(0, CHUNK2)
    def _zrow(i):
        msgA[i, :] = zero16

    @pl.loop(0, ROWS_PER_TILE // 64)
    def _zacc(i):
        pltpu.sync_copy(msgA.at[pl.ds(0, 64)],
                        acc.at[pl.ds(s * ROWS_PER_TILE + i * 64, 64)])

    plsc.subcore_barrier()

    base_w = wid * EDGES_PER_TILE2
    sets = ((sidxA, didxA, msgA, gA, sA),
            (sidxB, didxB, msgB, gB, sB),
            (sidxC, didxC, msgC, gC, sC))

    def load_and_gather(i, st):
        sidx, didx, msg, g, _ = st
        b = base_w + i * CHUNK2
        pltpu.sync_copy(src_hbm.at[pl.ds(b, CHUNK2)], sidx)
        pltpu.sync_copy(dst_hbm.at[pl.ds(b, CHUNK2)], didx)
        pltpu.async_copy(t2_hbm.at[sidx], msg, g)

    def do_chunk(st):
        sidx, didx, msg, g, sc = st
        pltpu.make_async_copy(t2_hbm.at[sidx], msg, g).wait()

        @plsc.parallel_loop(0, CHUNK2 // 16, 1, unroll=2)
        def _grp(g_):
            sv = sidx[pl.ds(g_ * 16, 16)]
            dv = didx[pl.ds(g_ * 16, 16)]
            av = plsc.load_gather(as2l, [sv])
            bv = plsc.load_gather(ad2l, [dv])
            w2 = _lrelu_exp(av + bv)
            for j in range(16):
                bc = _lane_bcast(w2, j)
                msg[g_ * 16 + j, :] = msg[g_ * 16 + j, :] * bc

        pltpu.async_copy(msg, acc.at[didx], sc, add=True)

    def wait_scatter(st):
        sidx, didx, msg, g, sc = st
        pltpu.make_async_copy(msg, acc.at[didx], sc).wait()

    load_and_gather(0, sets[0])
    load_and_gather(1, sets[1])
    do_chunk(sets[0])
    load_and_gather(2, sets[2])

    @pl.loop(0, (NCHUNK2 - 3) // 3)
    def _round(r):
        i = 1 + 3 * r
        do_chunk(sets[1])
        wait_scatter(sets[0])
        load_and_gather(i + 2, sets[0])
        do_chunk(sets[2])
        wait_scatter(sets[1])
        load_and_gather(i + 3, sets[1])
        do_chunk(sets[0])
        wait_scatter(sets[2])
        load_and_gather(i + 4, sets[2])

    do_chunk(sets[1])
    do_chunk(sets[2])
    wait_scatter(sets[0])
    wait_scatter(sets[1])
    wait_scatter(sets[2])

    plsc.subcore_barrier()
    row0 = s * ROWS_PER_TILE
    src_slice = acc.at[pl.ds(row0, ROWS_PER_TILE)]

    @pl.when(c == 0)
    def _w0():
        pltpu.sync_copy(src_slice, out_hbm.at[0, pl.ds(row0, ROWS_PER_TILE)])

    @pl.when(c == 1)
    def _w1():
        pltpu.sync_copy(src_slice, out_hbm.at[1, pl.ds(row0, ROWS_PER_TILE)])


def _sc2(src, dst, T2, as2f, ad2f):
    mesh = plsc.VectorSubcoreMesh(core_axis_name="c", subcore_axis_name="s")
    return pl.kernel(
        _sc2_body,
        mesh=mesh,
        compiler_params=pltpu.CompilerParams(use_tc_tiling_on_sc=False, needs_layout_passes=False),
        out_type=jax.ShapeDtypeStruct((2, NP, 16), jnp.float32),
        scratch_types=[
            pltpu.VMEM((CHUNK2,), jnp.int32),
            pltpu.VMEM((CHUNK2,), jnp.int32),
            pltpu.VMEM((CHUNK2,), jnp.int32),
            pltpu.VMEM((CHUNK2,), jnp.int32),
            pltpu.VMEM((CHUNK2,), jnp.int32),
            pltpu.VMEM((CHUNK2,), jnp.int32),
            pltpu.VMEM((CHUNK2, 16), jnp.float32),
            pltpu.VMEM((CHUNK2, 16), jnp.float32),
            pltpu.VMEM((CHUNK2, 16), jnp.float32),
            pltpu.VMEM((NP,), jnp.float32),
            pltpu.VMEM((NP,), jnp.float32),
            pltpu.VMEM_SHARED((NP, 16), jnp.float32),
            pltpu.SemaphoreType.DMA,
            pltpu.SemaphoreType.DMA,
            pltpu.SemaphoreType.DMA,
            pltpu.SemaphoreType.DMA,
            pltpu.SemaphoreType.DMA,
            pltpu.SemaphoreType.DMA,
        ],
    )(src, dst, T2, as2f, ad2f)


def _stage_c_body(p0_ref, p1_ref, t2_ref, a2f_ref, b2_ref, out_ref):
    acc2 = p0_ref[...] + p1_ref[...]
    a2f = a2f_ref[...]
    ws2 = _lrelu_exp(a2f[:, 0:1] + a2f[:, 1:2])
    numf = acc2 + ws2 * t2_ref[...]
    den2 = numf[:, 10:11]
    logits = numf / den2 + b2_ref[...]
    lane = lax.broadcasted_iota(jnp.int32, (1, 16), 1)
    mask = lane < 10
    lm = jnp.where(mask, logits, -1e30)
    m = jnp.max(lm, axis=1, keepdims=True)
    ex = jnp.where(mask, jnp.exp(lm - m), 0.0)
    out_ref[...] = lm - m - jnp.log(jnp.sum(ex, axis=1, keepdims=True))


def _stage_c(P20, P21, T2, A2f, b2m):
    mb = 2048
    grid = (NP // mb,)
    return pl.pallas_call(
        _stage_c_body,
        grid=grid,
        in_specs=[
            pl.BlockSpec((mb, 16), lambda i: (i, 0)),
            pl.BlockSpec((mb, 16), lambda i: (i, 0)),
            pl.BlockSpec((mb, 16), lambda i: (i, 0)),
            pl.BlockSpec((mb, 16), lambda i: (i, 0)),
            pl.BlockSpec((1, 16), lambda i: (0, 0)),
        ],
        out_specs=pl.BlockSpec((mb, 16), lambda i: (i, 0)),
        out_shape=jax.ShapeDtypeStruct((NP, 16), jnp.float32),
    )(P20, P21, T2, A2f, b2m)


def kernel(x, edge_index, W1, a1_src, a1_dst, b1, W2, a2_src, a2_dst, b2):
    pad = jnp.full((EP - E,), N, jnp.int32)
    src = jnp.concatenate([edge_index[0], pad])
    dst = jnp.concatenate([edge_index[1], pad])
    xp = jnp.pad(x, ((0, NP - N), (0, 0)))
    eye8 = jnp.eye(8, dtype=jnp.float32)
    A1s = jnp.pad((a1_src[:, :, None] * eye8[:, None, :]).reshape(D1, 8),
                  ((0, 0), (0, 8)))
    A1d = jnp.pad((a1_dst[:, :, None] * eye8[:, None, :]).reshape(D1, 8),
                  ((0, 0), (0, 8)))
    Rsel = jnp.repeat(jnp.eye(16, dtype=jnp.float32)[:, :8], 16, axis=1)
    W2p = jnp.pad(W2, ((0, 0), (0, 6)))
    A2W = jnp.concatenate(
        [jnp.pad(a2_src[0], (0, 6))[:, None], jnp.pad(a2_dst[0], (0, 6))[:, None],
         jnp.zeros((16, 14), jnp.float32)], axis=1)
    b1m = b1[None, :]
    b2m = jnp.pad(b2, (0, 6))[None, :]

    HX, AD = _stage_a(xp, W1, A1s, A1d)
    P = _sc1(src, dst, HX, AD)
    T2, A2f = _stage_b(P[0], P[1], HX, AD, b1m, W2p, A2W, Rsel)
    P2 = _sc2(src, dst, T2, A2f[:, 0], A2f[:, 1])
    C = _stage_c(P2[0], P2[1], T2, A2f, b2m)
    return C[:N, :10]


# speedup vs baseline: 1.6430x; 1.0478x over previous
import jax
import jax.numpy as jnp
from jax import lax
from jax.experimental import pallas as pl
from jax.experimental.pallas import tpu as pltpu
from jax.experimental.pallas import tpu_sc as plsc

N = 10000
NP = 10240
E = 320000
EP = 322560
FIN = 128
D1 = 128
ACC_W = 144
NCORES = 2
NSUB = 16
NTILES = NCORES * NSUB
CHUNK1 = 80
EPT_C0 = 11760
EPT_C1 = 8400
NC0 = EPT_C0 // CHUNK1
NC1 = EPT_C1 // CHUNK1
EDGES_PER_TILE2 = EP // NTILES
CHUNK2 = 112
NCHUNK2 = EDGES_PER_TILE2 // CHUNK2
ROWS_PER_TILE = NP // NSUB

_HI = jax.lax.Precision.HIGHEST


def _lrelu_exp(v):
    return jnp.exp(jnp.maximum(v, 0.0) + 0.2 * jnp.minimum(v, 0.0))


def _lane_bcast(v, lane):
    idx = jnp.full((16, 1), lane, jnp.int32)
    dn = lax.GatherDimensionNumbers(
        offset_dims=(), collapsed_slice_dims=(0,), start_index_map=(0,))
    return lax.gather(v, idx, dn, (1,),
                      mode=lax.GatherScatterMode.PROMISE_IN_BOUNDS)


def _stage_a_body(x_ref, w1_ref, a1s_ref, a1d_ref, hx_ref, ad_ref):
    h = lax.dot_general(x_ref[...], w1_ref[...], (((1,), (0,)), ((), ())),
                        precision=_HI, preferred_element_type=jnp.float32)
    hx_ref[:, :D1] = h
    hx_ref[:, D1:ACC_W] = lax.dot_general(
        h, a1s_ref[...], (((1,), (0,)), ((), ())),
        precision=_HI, preferred_element_type=jnp.float32)
    ad_ref[...] = lax.dot_general(h, a1d_ref[...], (((1,), (0,)), ((), ())),
                                  precision=_HI,
                                  preferred_element_type=jnp.float32)


def _stage_a(xp, W1, A1s, A1d):
    mb = 2048
    grid = (NP // mb,)
    return pl.pallas_call(
        _stage_a_body,
        grid=grid,
        in_specs=[
            pl.BlockSpec((mb, FIN), lambda i: (i, 0)),
            pl.BlockSpec((FIN, D1), lambda i: (0, 0)),
            pl.BlockSpec((D1, 16), lambda i: (0, 0)),
            pl.BlockSpec((D1, 16), lambda i: (0, 0)),
        ],
        out_specs=[
            pl.BlockSpec((mb, ACC_W), lambda i: (i, 0)),
            pl.BlockSpec((mb, 16), lambda i: (i, 0)),
        ],
        out_shape=[
            jax.ShapeDtypeStruct((NP, ACC_W), jnp.float32),
            jax.ShapeDtypeStruct((NP, 16), jnp.float32),
        ],
    )(xp, W1, A1s, A1d)


def _sc1_body(src_hbm, dst_hbm, hx_hbm, ad_hbm, out0_hbm, out1_hbm,
              sidxA, didxA, sidxB, didxB, sidxC, didxC,
              adrA, adrB, adrC, msgA, msgB, msgC,
              acc, gA, gB, gC, sA, sB, sC):
    c = lax.axis_index("c")
    s = lax.axis_index("s")
    zero16 = jnp.zeros((16,), jnp.float32)

    @pl.loop(0, CHUNK1)
    def _zrow(i):
        @pl.loop(0, ACC_W // 16)
        def _zcol(j):
            msgA[i, pl.ds(j * 16, 16)] = zero16

    @pl.loop(0, ROWS_PER_TILE // CHUNK1)
    def _zacc(i):
        pltpu.sync_copy(msgA, acc.at[pl.ds(s * ROWS_PER_TILE + i * CHUNK1,
                                           CHUNK1)])

    plsc.subcore_barrier()

    sets = ((sidxA, didxA, adrA, msgA, gA, sA),
            (sidxB, didxB, adrB, msgB, gB, sB),
            (sidxC, didxC, adrC, msgC, gC, sC))

    def pipeline(base_w, nchunk):
        def load_and_gather(i, st):
            sidx, didx, adr, msg, g, _ = st
            b = base_w + i * CHUNK1
            pltpu.sync_copy(src_hbm.at[pl.ds(b, CHUNK1)], sidx)
            pltpu.sync_copy(dst_hbm.at[pl.ds(b, CHUNK1)], didx)
            pltpu.async_copy(hx_hbm.at[sidx], msg, g)
            pltpu.async_copy(ad_hbm.at[didx], adr, g)

        def do_chunk(st):
            sidx, didx, adr, msg, g, sc = st
            pltpu.make_async_copy(hx_hbm.at[sidx], msg, g).wait()
            pltpu.make_async_copy(ad_hbm.at[didx], adr, g).wait()

            @plsc.parallel_loop(0, CHUNK1, 1, unroll=2)
            def _edge(e):
                v = msg[e, pl.ds(D1, 16)] + adr[e, :]
                w = _lrelu_exp(v)
                msg[e, pl.ds(D1, 16)] = w
                for hh in range(8):
                    bc = _lane_bcast(w, hh)
                    msg[e, pl.ds(hh * 16, 16)] = (
                        msg[e, pl.ds(hh * 16, 16)] * bc)

            pltpu.async_copy(msg, acc.at[didx], sc, add=True)

        def wait_scatter(st):
            sidx, didx, adr, msg, g, sc = st
            pltpu.make_async_copy(msg, acc.at[didx], sc).wait()

        load_and_gather(0, sets[0])
        load_and_gather(1, sets[1])
        do_chunk(sets[0])
        load_and_gather(2, sets[2])

        @pl.loop(0, (nchunk - 3) // 3)
        def _round(r):
            i = 1 + 3 * r
            do_chunk(sets[1])
            wait_scatter(sets[0])
            load_and_gather(i + 2, sets[0])
            do_chunk(sets[2])
            wait_scatter(sets[1])
            load_and_gather(i + 3, sets[1])
            do_chunk(sets[0])
            wait_scatter(sets[2])
            load_and_gather(i + 4, sets[2])

        do_chunk(sets[1])
        do_chunk(sets[2])
        wait_scatter(sets[0])
        wait_scatter(sets[1])
        wait_scatter(sets[2])

    @pl.when(c == 0)
    def _p0():
        pipeline(s * EPT_C0, NC0)

    @pl.when(c == 1)
    def _p1():
        pipeline(EPT_C0 * NSUB + s * EPT_C1, NC1)

    plsc.subcore_barrier()
    row0 = s * ROWS_PER_TILE
    src_slice = acc.at[pl.ds(row0, ROWS_PER_TILE)]

    @pl.when(c == 0)
    def _w0():
        pltpu.sync_copy(src_slice, out0_hbm.at[pl.ds(row0, ROWS_PER_TILE)])

    @pl.when(c == 1)
    def _w1():
        pltpu.sync_copy(src_slice, out1_hbm.at[pl.ds(row0, ROWS_PER_TILE)])


def _sc1(src, dst, HX, AD):
    mesh = plsc.VectorSubcoreMesh(core_axis_name="c", subcore_axis_name="s")
    return pl.kernel(
        _sc1_body,
        mesh=mesh,
        compiler_params=pltpu.CompilerParams(use_tc_tiling_on_sc=False, needs_layout_passes=False),
        out_type=[jax.ShapeDtypeStruct((NP, ACC_W), jnp.float32),
                  jax.ShapeDtypeStruct((NP, ACC_W), jnp.float32)],
        scratch_types=[
            pltpu.VMEM((CHUNK1,), jnp.int32),
            pltpu.VMEM((CHUNK1,), jnp.int32),
            pltpu.VMEM((CHUNK1,), jnp.int32),
            pltpu.VMEM((CHUNK1,), jnp.int32),
            pltpu.VMEM((CHUNK1,), jnp.int32),
            pltpu.VMEM((CHUNK1,), jnp.int32),
            pltpu.VMEM((CHUNK1, 16), jnp.float32),
            pltpu.VMEM((CHUNK1, 16), jnp.float32),
            pltpu.VMEM((CHUNK1, 16), jnp.float32),
            pltpu.VMEM((CHUNK1, ACC_W), jnp.float32),
            pltpu.VMEM((CHUNK1, ACC_W), jnp.float32),
            pltpu.VMEM((CHUNK1, ACC_W), jnp.float32),
            pltpu.VMEM_SHARED((NP, ACC_W), jnp.float32),
            pltpu.SemaphoreType.DMA,
            pltpu.SemaphoreType.DMA,
            pltpu.SemaphoreType.DMA,
            pltpu.SemaphoreType.DMA,
            pltpu.SemaphoreType.DMA,
            pltpu.SemaphoreType.DMA,
        ],
    )(src, dst, HX, AD)


def _stage_b_body(p0_ref, p1_ref, hx_ref, ad_ref, b1_ref, w2_ref,
                  a2w_ref, rsel_ref, t2_ref, a2f_ref):
    accf = p0_ref[...] + p1_ref[...]
    hx = hx_ref[...]
    ws = _lrelu_exp(hx[:, D1:ACC_W] + ad_ref[...])
    rsel = rsel_ref[...]
    ws_wide = lax.dot_general(ws, rsel, (((1,), (0,)), ((), ())),
                              precision=_HI, preferred_element_type=jnp.float32)
    den_wide = lax.dot_general(accf[:, D1:ACC_W] + ws, rsel,
                               (((1,), (0,)), ((), ())),
                               precision=_HI, preferred_element_type=jnp.float32)
    num = accf[:, :D1] + ws_wide * hx[:, :D1]
    out1 = num / den_wide + b1_ref[...]
    z = jnp.where(out1 > 0, out1, jnp.exp(jnp.minimum(out1, 0.0)) - 1.0)
    h2 = lax.dot_general(z, w2_ref[...], (((1,), (0,)), ((), ())),
                         precision=_HI, preferred_element_type=jnp.float32)
    lane = lax.broadcasted_iota(jnp.int32, (1, 16), 1)
    t2_ref[...] = h2 + jnp.where(lane == 10, 1.0, 0.0)
    a2f_ref[...] = lax.dot_general(h2, a2w_ref[...], (((1,), (0,)), ((), ())),
                                   precision=_HI,
                                   preferred_element_type=jnp.float32)


def _stage_b(P0, P1, HX, AD, b1m, W2p, A2W, Rsel):
    mb = 2048
    grid = (NP // mb,)
    return pl.pallas_call(
        _stage_b_body,
        grid=grid,
        in_specs=[
            pl.BlockSpec((mb, ACC_W), lambda i: (i, 0)),
            pl.BlockSpec((mb, ACC_W), lambda i: (i, 0)),
            pl.BlockSpec((mb, ACC_W), lambda i: (i, 0)),
            pl.BlockSpec((mb, 16), lambda i: (i, 0)),
            pl.BlockSpec((1, D1), lambda i: (0, 0)),
            pl.BlockSpec((D1, 16), lambda i: (0, 0)),
            pl.BlockSpec((16, 16), lambda i: (0, 0)),
            pl.BlockSpec((16, D1), lambda i: (0, 0)),
        ],
        out_specs=[
            pl.BlockSpec((mb, 16), lambda i: (i, 0)),
            pl.BlockSpec((mb, 16), lambda i: (i, 0)),
        ],
        out_shape=[
            jax.ShapeDtypeStruct((NP, 16), jnp.float32),
            jax.ShapeDtypeStruct((NP, 16), jnp.float32),
        ],
    )(P0, P1, HX, AD, b1m, W2p, A2W, Rsel)


def _sc2_body(src_hbm, dst_hbm, t2_hbm, as2_hbm, ad2_hbm, out0_hbm, out1_hbm,
              sidxA, didxA, sidxB, didxB, sidxC, didxC,
              msgA, msgB, msgC, as2l, ad2l,
              acc, gA, gB, gC, sA, sB, sC):
    c = lax.axis_index("c")
    s = lax.axis_index("s")
    wid = c * NSUB + s
    pltpu.sync_copy(as2_hbm, as2l)
    pltpu.sync_copy(ad2_hbm, ad2l)
    zero16 = jnp.zeros((16,), jnp.float32)

    @pl.loop(0, CHUNK2)
    def _zrow(i):
        msgA[i, :] = zero16

    @pl.loop(0, ROWS_PER_TILE // 64)
    def _zacc(i):
        pltpu.sync_copy(msgA.at[pl.ds(0, 64)],
                        acc.at[pl.ds(s * ROWS_PER_TILE + i * 64, 64)])

    plsc.subcore_barrier()

    base_w = wid * EDGES_PER_TILE2
    sets = ((sidxA, didxA, msgA, gA, sA),
            (sidxB, didxB, msgB, gB, sB),
            (sidxC, didxC, msgC, gC, sC))

    def load_and_gather(i, st):
        sidx, didx, msg, g, _ = st
        b = base_w + i * CHUNK2
        pltpu.sync_copy(src_hbm.at[pl.ds(b, CHUNK2)], sidx)
        pltpu.sync_copy(dst_hbm.at[pl.ds(b, CHUNK2)], didx)
        pltpu.async_copy(t2_hbm.at[sidx], msg, g)

    def do_chunk(st):
        sidx, didx, msg, g, sc = st
        pltpu.make_async_copy(t2_hbm.at[sidx], msg, g).wait()

        @plsc.parallel_loop(0, CHUNK2 // 16, 1, unroll=2)
        def _grp(g_):
            sv = sidx[pl.ds(g_ * 16, 16)]
            dv = didx[pl.ds(g_ * 16, 16)]
            av = plsc.load_gather(as2l, [sv])
            bv = plsc.load_gather(ad2l, [dv])
            w2 = _lrelu_exp(av + bv)
            for j in range(16):
                bc = _lane_bcast(w2, j)
                msg[g_ * 16 + j, :] = msg[g_ * 16 + j, :] * bc

        pltpu.async_copy(msg, acc.at[didx], sc, add=True)

    def wait_scatter(st):
        sidx, didx, msg, g, sc = st
        pltpu.make_async_copy(msg, acc.at[didx], sc).wait()

    load_and_gather(0, sets[0])
    load_and_gather(1, sets[1])
    do_chunk(sets[0])
    load_and_gather(2, sets[2])

    @pl.loop(0, (NCHUNK2 - 3) // 3)
    def _round(r):
        i = 1 + 3 * r
        do_chunk(sets[1])
        wait_scatter(sets[0])
        load_and_gather(i + 2, sets[0])
        do_chunk(sets[2])
        wait_scatter(sets[1])
        load_and_gather(i + 3, sets[1])
        do_chunk(sets[0])
        wait_scatter(sets[2])
        load_and_gather(i + 4, sets[2])

    do_chunk(sets[1])
    do_chunk(sets[2])
    wait_scatter(sets[0])
    wait_scatter(sets[1])
    wait_scatter(sets[2])

    plsc.subcore_barrier()
    row0 = s * ROWS_PER_TILE
    src_slice = acc.at[pl.ds(row0, ROWS_PER_TILE)]

    @pl.when(c == 0)
    def _w0():
        pltpu.sync_copy(src_slice, out0_hbm.at[pl.ds(row0, ROWS_PER_TILE)])

    @pl.when(c == 1)
    def _w1():
        pltpu.sync_copy(src_slice, out1_hbm.at[pl.ds(row0, ROWS_PER_TILE)])


def _sc2(src, dst, T2, as2f, ad2f):
    mesh = plsc.VectorSubcoreMesh(core_axis_name="c", subcore_axis_name="s")
    return pl.kernel(
        _sc2_body,
        mesh=mesh,
        compiler_params=pltpu.CompilerParams(use_tc_tiling_on_sc=False, needs_layout_passes=False),
        out_type=[jax.ShapeDtypeStruct((NP, 16), jnp.float32),
                  jax.ShapeDtypeStruct((NP, 16), jnp.float32)],
        scratch_types=[
            pltpu.VMEM((CHUNK2,), jnp.int32),
            pltpu.VMEM((CHUNK2,), jnp.int32),
            pltpu.VMEM((CHUNK2,), jnp.int32),
            pltpu.VMEM((CHUNK2,), jnp.int32),
            pltpu.VMEM((CHUNK2,), jnp.int32),
            pltpu.VMEM((CHUNK2,), jnp.int32),
            pltpu.VMEM((CHUNK2, 16), jnp.float32),
            pltpu.VMEM((CHUNK2, 16), jnp.float32),
            pltpu.VMEM((CHUNK2, 16), jnp.float32),
            pltpu.VMEM((NP,), jnp.float32),
            pltpu.VMEM((NP,), jnp.float32),
            pltpu.VMEM_SHARED((NP, 16), jnp.float32),
            pltpu.SemaphoreType.DMA,
            pltpu.SemaphoreType.DMA,
            pltpu.SemaphoreType.DMA,
            pltpu.SemaphoreType.DMA,
            pltpu.SemaphoreType.DMA,
            pltpu.SemaphoreType.DMA,
        ],
    )(src, dst, T2, as2f, ad2f)


def _stage_c_body(p0_ref, p1_ref, t2_ref, a2f_ref, b2_ref, out_ref):
    acc2 = p0_ref[...] + p1_ref[...]
    a2f = a2f_ref[...]
    ws2 = _lrelu_exp(a2f[:, 0:1] + a2f[:, 8:9])
    numf = acc2 + ws2 * t2_ref[...]
    den2 = numf[:, 10:11]
    logits = numf / den2 + b2_ref[...]
    lane = lax.broadcasted_iota(jnp.int32, (1, 16), 1)
    mask = lane < 10
    lm = jnp.where(mask, logits, -1e30)
    m = jnp.max(lm, axis=1, keepdims=True)
    ex = jnp.where(mask, jnp.exp(lm - m), 0.0)
    out_ref[...] = lm - m - jnp.log(jnp.sum(ex, axis=1, keepdims=True))


def _stage_c(P20, P21, T2, A2f, b2m):
    mb = 2048
    grid = (NP // mb,)
    return pl.pallas_call(
        _stage_c_body,
        grid=grid,
        in_specs=[
            pl.BlockSpec((mb, 16), lambda i: (i, 0)),
            pl.BlockSpec((mb, 16), lambda i: (i, 0)),
            pl.BlockSpec((mb, 16), lambda i: (i, 0)),
            pl.BlockSpec((mb, 16), lambda i: (i, 0)),
            pl.BlockSpec((1, 16), lambda i: (0, 0)),
        ],
        out_specs=pl.BlockSpec((mb, 16), lambda i: (i, 0)),
        out_shape=jax.ShapeDtypeStruct((NP, 16), jnp.float32),
    )(P20, P21, T2, A2f, b2m)


def kernel(x, edge_index, W1, a1_src, a1_dst, b1, W2, a2_src, a2_dst, b2):
    pad = jnp.full((EP - E,), N, jnp.int32)
    src = jnp.concatenate([edge_index[0], pad])
    dst = jnp.concatenate([edge_index[1], pad])
    xp = jnp.pad(x, ((0, NP - N), (0, 0)))
    eye8 = jnp.eye(8, dtype=jnp.float32)
    A1s = jnp.pad((a1_src[:, :, None] * eye8[:, None, :]).reshape(D1, 8),
                  ((0, 0), (0, 8)))
    A1d = jnp.pad((a1_dst[:, :, None] * eye8[:, None, :]).reshape(D1, 8),
                  ((0, 0), (0, 8)))
    Rsel = jnp.repeat(jnp.eye(16, dtype=jnp.float32)[:, :8], 16, axis=1)
    W2p = jnp.pad(W2, ((0, 0), (0, 6)))
    A2W = jnp.concatenate(
        [jnp.pad(a2_src[0], (0, 6))[:, None], jnp.zeros((16, 7), jnp.float32),
         jnp.pad(a2_dst[0], (0, 6))[:, None], jnp.zeros((16, 7), jnp.float32)],
        axis=1)
    b1m = b1[None, :]
    b2m = jnp.pad(b2, (0, 6))[None, :]

    HX, AD = _stage_a(xp, W1, A1s, A1d)
    P0, P1 = _sc1(src, dst, HX, AD)
    T2, A2f = _stage_b(P0, P1, HX, AD, b1m, W2p, A2W, Rsel)
    P20, P21 = _sc2(src, dst, T2, A2f[:, 0], A2f[:, 8])
    C = _stage_c(P20, P21, T2, A2f, b2m)
    return C[:N, :10]
